# Initial kernel scaffold; baseline (speedup 1.0000x reference)
import functools

import jax
import jax.numpy as jnp
from jax import lax
from jax.experimental import pallas as pl
from jax.experimental.pallas import tpu as pltpu
from jax.experimental.pallas import tpu_sc as plsc

B, A = 16384, 8
E = B * A
NBUF = 64
DT = 0.001
MAX_DELAY = 0.06
K = 17

NC, NS, LANES = 2, 16, 16
NW = NC * NS
EPW = E // NW
NVREG = EPW // LANES
CHUNK = 128
NSTREAM = EPW // CHUNK

_mesh = plsc.VectorSubcoreMesh(core_axis_name="c", subcore_axis_name="s")


@functools.partial(
    pl.kernel,
    out_type=jax.ShapeDtypeStruct((E,), jnp.float32),
    mesh=_mesh,
    scratch_types=[
        pltpu.VMEM((EPW,), jnp.float32),
        pltpu.VMEM((EPW,), jnp.float32),
        pltpu.VMEM((EPW,), jnp.float32),
        pltpu.VMEM((EPW,), jnp.float32),
        pltpu.VMEM((EPW,), jnp.float32),
        pltpu.VMEM((EPW,), jnp.int32),
        pltpu.VMEM((EPW,), jnp.int32),
        pltpu.VMEM((EPW,), jnp.float32),
        pltpu.VMEM((EPW,), jnp.float32),
        pltpu.VMEM((K, K), jnp.float32),
        pltpu.VMEM((K, K), jnp.float32),
        pltpu.VMEM((16,), jnp.float32),
        pltpu.VMEM((16,), jnp.int32),
        pltpu.SemaphoreType.DMA,
    ],
)
def _pam_sc(tgt_hbm, cur_hbm, buf_hbm, tau_hbm, dead_hbm, pf_hbm, pi_hbm,
            out_hbm, tgt_v, cur_v, out_v, frac_v, alpha_v, idx0_v, idx1_v,
            g0_v, g1_v, tau_v, dead_v, pf_v, pi_v, sem):
    wid = lax.axis_index("s") * NC + lax.axis_index("c")
    base = wid * EPW
    pltpu.sync_copy(tgt_hbm.at[pl.ds(base, EPW)], tgt_v)
    pltpu.sync_copy(cur_hbm.at[pl.ds(base, EPW)], cur_v)
    pltpu.sync_copy(tau_hbm, tau_v)
    pltpu.sync_copy(dead_hbm, dead_v)
    pltpu.sync_copy(pf_hbm, pf_v)
    pltpu.sync_copy(pi_hbm, pi_v)

    lo = pf_v[0]
    hi = pf_v[1]
    inv_dx = pf_v[2]
    wi_f = pf_v[3]
    wcol = pi_v[0]
    lane = lax.iota(jnp.int32, LANES)

    def compute(i, _):
        off = pl.multiple_of(i * LANES, LANES)
        tg = tgt_v[pl.ds(off, LANES)]
        cu = cur_v[pl.ds(off, LANES)]
        tx = (jnp.minimum(jnp.maximum(tg, lo), hi) - lo) * inv_dx
        ty = (jnp.minimum(jnp.maximum(cu, lo), hi) - lo) * inv_dx
        ix = jnp.minimum(tx.astype(jnp.int32), K - 2)
        iy = jnp.minimum(ty.astype(jnp.int32), K - 2)
        wx = tx - ix.astype(jnp.float32)
        wy = ty - iy.astype(jnp.float32)
        ixp = ix + 1
        iyp = iy + 1
        w00 = (1.0 - wx) * (1.0 - wy)
        w10 = wx * (1.0 - wy)
        w01 = (1.0 - wx) * wy
        w11 = wx * wy
        tau = (plsc.load_gather(tau_v, [ix, iy]) * w00
               + plsc.load_gather(tau_v, [ixp, iy]) * w10
               + plsc.load_gather(tau_v, [ix, iyp]) * w01
               + plsc.load_gather(tau_v, [ixp, iyp]) * w11)
        dead = (plsc.load_gather(dead_v, [ix, iy]) * w00
                + plsc.load_gather(dead_v, [ixp, iy]) * w10
                + plsc.load_gather(dead_v, [ix, iyp]) * w01
                + plsc.load_gather(dead_v, [ixp, iyp]) * w11)
        alpha = 1.0 - jnp.exp(-DT / jnp.maximum(tau, 1e-6))
        d = jnp.minimum(jnp.maximum(dead, 0.0), MAX_DELAY) / DT
        rp = wi_f - d
        i0 = rp.astype(jnp.int32)
        frac = rp - i0.astype(jnp.float32)
        c0 = jnp.bitwise_and(i0, NBUF - 1)
        c1 = jnp.bitwise_and(i0 + 1, NBUF - 1)
        e16 = (base + off) + lane
        idx0_v[pl.ds(off, LANES)] = e16 * NBUF + c0
        idx1_v[pl.ds(off, LANES)] = e16 * NBUF + c1
        frac_v[pl.ds(off, LANES)] = frac
        alpha_v[pl.ds(off, LANES)] = alpha
        return 0

    lax.fori_loop(0, NVREG, compute, 0)

    copies = []
    for j in range(NSTREAM):
        sl = pl.ds(j * CHUNK, CHUNK)
        copies.append(pltpu.async_copy(buf_hbm.at[idx0_v.at[sl]], g0_v.at[sl], sem))
        copies.append(pltpu.async_copy(buf_hbm.at[idx1_v.at[sl]], g1_v.at[sl], sem))
    for c in copies:
        c.wait()

    def blend(i, _):
        off = pl.multiple_of(i * LANES, LANES)
        tg = tgt_v[pl.ds(off, LANES)]
        cu = cur_v[pl.ds(off, LANES)]
        fr = frac_v[pl.ds(off, LANES)]
        al = alpha_v[pl.ds(off, LANES)]
        c0 = jnp.bitwise_and(idx0_v[pl.ds(off, LANES)], NBUF - 1)
        c1 = jnp.bitwise_and(idx1_v[pl.ds(off, LANES)], NBUF - 1)
        s0 = jnp.where(c0 == wcol, tg, g0_v[pl.ds(off, LANES)])
        s1 = jnp.where(c1 == wcol, tg, g1_v[pl.ds(off, LANES)])
        s = s0 * (1.0 - fr) + s1 * fr
        out_v[pl.ds(off, LANES)] = cu + al * (s - cu)
        return 0

    lax.fori_loop(0, NVREG, blend, 0)
    pltpu.sync_copy(out_v, out_hbm.at[pl.ds(base, EPW)])


def kernel(target_pressure, buf, current_pressure, p_axis, tau_table,
           dead_table, write_idx):
    tgt = target_pressure.reshape(E)
    cur = current_pressure.reshape(E)
    bufflat = buf.reshape(E * NBUF)
    lo = p_axis[0]
    hi = p_axis[K - 1]
    inv_dx = (K - 1) / (hi - lo)
    wi_f = write_idx.astype(jnp.float32)
    pf = jnp.concatenate([jnp.stack([lo, hi, inv_dx, wi_f]),
                          jnp.zeros((12,), jnp.float32)])
    wcol = jnp.mod(write_idx, NBUF)
    pi = jnp.concatenate([wcol[None].astype(jnp.int32),
                          jnp.zeros((15,), jnp.int32)])
    out = _pam_sc(tgt, cur, bufflat, tau_table, dead_table, pf, pi)
    return out.reshape(B, A)


# speedup vs baseline: 82.2924x; 82.2924x over previous
import functools

import jax
import jax.numpy as jnp
from jax import lax
from jax.experimental import pallas as pl
from jax.experimental.pallas import tpu as pltpu
from jax.experimental.pallas import tpu_sc as plsc

B, A = 16384, 8
E = B * A
NBUF = 64
DT = 0.001
MAX_DELAY = 0.06
K = 17

NC, NS, LANES = 2, 16, 16
NW = NC * NS
EPW = E // NW
NVREG = EPW // LANES
CHUNK = 128
NSTREAM = EPW // CHUNK

_mesh = plsc.VectorSubcoreMesh(core_axis_name="c", subcore_axis_name="s")


@functools.partial(
    pl.kernel,
    out_type=jax.ShapeDtypeStruct((E,), jnp.float32),
    mesh=_mesh,
    compiler_params=pltpu.CompilerParams(needs_layout_passes=False),
    scratch_types=[
        pltpu.VMEM((EPW,), jnp.float32),
        pltpu.VMEM((EPW,), jnp.float32),
        pltpu.VMEM((EPW,), jnp.float32),
        pltpu.VMEM((EPW,), jnp.float32),
        pltpu.VMEM((EPW,), jnp.float32),
        pltpu.VMEM((EPW,), jnp.int32),
        pltpu.VMEM((EPW,), jnp.int32),
        pltpu.VMEM((EPW,), jnp.float32),
        pltpu.VMEM((EPW,), jnp.float32),
        pltpu.VMEM((K * K,), jnp.float32),
        pltpu.VMEM((K * K,), jnp.float32),
        pltpu.VMEM((16,), jnp.float32),
        pltpu.VMEM((16,), jnp.int32),
        pltpu.SemaphoreType.DMA,
    ],
)
def _pam_sc(tgt_hbm, cur_hbm, buf_hbm, tau_hbm, dead_hbm, pf_hbm, pi_hbm,
            out_hbm, tgt_v, cur_v, out_v, frac_v, alpha_v, idx0_v, idx1_v,
            g0_v, g1_v, tau_v, dead_v, pf_v, pi_v, sem):
    wid = lax.axis_index("s") * NC + lax.axis_index("c")
    base = wid * EPW
    pltpu.sync_copy(tgt_hbm.at[pl.ds(base, EPW)], tgt_v)
    pltpu.sync_copy(cur_hbm.at[pl.ds(base, EPW)], cur_v)
    pltpu.sync_copy(tau_hbm, tau_v)
    pltpu.sync_copy(dead_hbm, dead_v)
    pltpu.sync_copy(pf_hbm, pf_v)
    pltpu.sync_copy(pi_hbm, pi_v)

    pfv = pf_v[...]
    piv = pi_v[...]
    lo = pfv[0]
    hi = pfv[1]
    inv_dx = pfv[2]
    wi_f = pfv[3]
    wcol = piv[0]
    lane = lax.iota(jnp.int32, LANES)

    def compute(i, _):
        off = pl.multiple_of(i * LANES, LANES)
        tg = tgt_v[pl.ds(off, LANES)]
        cu = cur_v[pl.ds(off, LANES)]
        tx = (jnp.minimum(jnp.maximum(tg, lo), hi) - lo) * inv_dx
        ty = (jnp.minimum(jnp.maximum(cu, lo), hi) - lo) * inv_dx
        ix = jnp.minimum(tx.astype(jnp.int32), K - 2)
        iy = jnp.minimum(ty.astype(jnp.int32), K - 2)
        wx = tx - ix.astype(jnp.float32)
        wy = ty - iy.astype(jnp.float32)
        w00 = (1.0 - wx) * (1.0 - wy)
        w10 = wx * (1.0 - wy)
        w01 = (1.0 - wx) * wy
        w11 = wx * wy
        q00 = ix * K + iy
        q10 = q00 + K
        q01 = q00 + 1
        q11 = q00 + K + 1
        tau = (plsc.load_gather(tau_v, [q00]) * w00
               + plsc.load_gather(tau_v, [q10]) * w10
               + plsc.load_gather(tau_v, [q01]) * w01
               + plsc.load_gather(tau_v, [q11]) * w11)
        dead = (plsc.load_gather(dead_v, [q00]) * w00
                + plsc.load_gather(dead_v, [q10]) * w10
                + plsc.load_gather(dead_v, [q01]) * w01
                + plsc.load_gather(dead_v, [q11]) * w11)
        alpha = 1.0 - jnp.exp(-DT / jnp.maximum(tau, 1e-6))
        d = jnp.minimum(jnp.maximum(dead, 0.0), MAX_DELAY) / DT
        rp = wi_f - d
        i0 = rp.astype(jnp.int32)
        frac = rp - i0.astype(jnp.float32)
        c0 = jnp.bitwise_and(i0, NBUF - 1)
        c1 = jnp.bitwise_and(i0 + 1, NBUF - 1)
        e16 = (base + off) + lane
        idx0_v[pl.ds(off, LANES)] = e16 * NBUF + c0
        idx1_v[pl.ds(off, LANES)] = e16 * NBUF + c1
        frac_v[pl.ds(off, LANES)] = frac
        alpha_v[pl.ds(off, LANES)] = alpha
        return 0

    lax.fori_loop(0, NVREG, compute, 0)

    copies = []
    for j in range(NSTREAM):
        sl = pl.ds(j * CHUNK, CHUNK)
        copies.append(pltpu.async_copy(buf_hbm.at[idx0_v.at[sl]], g0_v.at[sl], sem))
        copies.append(pltpu.async_copy(buf_hbm.at[idx1_v.at[sl]], g1_v.at[sl], sem))
    for c in copies:
        c.wait()

    def blend(i, _):
        off = pl.multiple_of(i * LANES, LANES)
        tg = tgt_v[pl.ds(off, LANES)]
        cu = cur_v[pl.ds(off, LANES)]
        fr = frac_v[pl.ds(off, LANES)]
        al = alpha_v[pl.ds(off, LANES)]
        c0 = jnp.bitwise_and(idx0_v[pl.ds(off, LANES)], NBUF - 1)
        c1 = jnp.bitwise_and(idx1_v[pl.ds(off, LANES)], NBUF - 1)
        s0 = jnp.where(c0 == wcol, tg, g0_v[pl.ds(off, LANES)])
        s1 = jnp.where(c1 == wcol, tg, g1_v[pl.ds(off, LANES)])
        s = s0 * (1.0 - fr) + s1 * fr
        out_v[pl.ds(off, LANES)] = cu + al * (s - cu)
        return 0

    lax.fori_loop(0, NVREG, blend, 0)
    pltpu.sync_copy(out_v, out_hbm.at[pl.ds(base, EPW)])


def kernel(target_pressure, buf, current_pressure, p_axis, tau_table,
           dead_table, write_idx):
    tgt = target_pressure.reshape(E)
    cur = current_pressure.reshape(E)
    bufflat = buf.reshape(E * NBUF)
    lo = p_axis[0]
    hi = p_axis[K - 1]
    inv_dx = (K - 1) / (hi - lo)
    wi_f = write_idx.astype(jnp.float32)
    pf = jnp.concatenate([jnp.stack([lo, hi, inv_dx, wi_f]),
                          jnp.zeros((12,), jnp.float32)])
    wcol = jnp.mod(write_idx, NBUF)
    pi = jnp.concatenate([wcol[None].astype(jnp.int32),
                          jnp.zeros((15,), jnp.int32)])
    out = _pam_sc(tgt, cur, bufflat, tau_table.reshape(K * K),
                  dead_table.reshape(K * K), pf, pi)
    return out.reshape(B, A)


# speedup vs baseline: 94.2549x; 1.1454x over previous
import functools

import jax
import jax.numpy as jnp
from jax import lax
from jax.experimental import pallas as pl
from jax.experimental.pallas import tpu as pltpu
from jax.experimental.pallas import tpu_sc as plsc

B, A = 16384, 8
NBUF = 64
DT = 0.001
MAX_DELAY = 0.06
K = 17

NC, NS, LANES = 2, 16, 16
NW = NC * NS
RPW = B // NW
CB = 32
NCHUNK = RPW // CB
CVREG = CB * A // LANES

_mesh = plsc.VectorSubcoreMesh(core_axis_name="c", subcore_axis_name="s")


@functools.partial(
    pl.kernel,
    out_type=jax.ShapeDtypeStruct((B, A), jnp.float32),
    mesh=_mesh,
    compiler_params=pltpu.CompilerParams(needs_layout_passes=False),
    scratch_types=[
        pltpu.VMEM((CB, A, NBUF), jnp.float32),
        pltpu.VMEM((CB, A, NBUF), jnp.float32),
        pltpu.VMEM((CB, A), jnp.float32),
        pltpu.VMEM((CB, A), jnp.float32),
        pltpu.VMEM((CB, A), jnp.float32),
        pltpu.VMEM((CB, A), jnp.float32),
        pltpu.VMEM((CB, A), jnp.float32),
        pltpu.VMEM((K, K), jnp.float32),
        pltpu.VMEM((K, K), jnp.float32),
        pltpu.VMEM((16,), jnp.float32),
        pltpu.VMEM((16,), jnp.int32),
        pltpu.SemaphoreType.DMA,
        pltpu.SemaphoreType.DMA,
        pltpu.SemaphoreType.DMA,
    ],
)
def _pam_sc(tgt_hbm, cur_hbm, buf_hbm, tau_hbm, dead_hbm, pf_hbm, pi_hbm,
            out_hbm, slab_v, slab_w, tgt_v, tgt_w, cur_v, cur_w, out_v,
            tau_v, dead_v, pf_v, pi_v, sem_a, sem_b, sem_o):
    wid = lax.axis_index("s") * NC + lax.axis_index("c")
    row0 = wid * RPW
    pltpu.sync_copy(tau_hbm, tau_v)
    pltpu.sync_copy(dead_hbm, dead_v)
    pltpu.sync_copy(pf_hbm, pf_v)
    pltpu.sync_copy(pi_hbm, pi_v)

    pfv = pf_v[...]
    piv = pi_v[...]
    lo = pfv[0]
    hi = pfv[1]
    inv_dx = pfv[2]
    wi_f = pfv[3]
    wcol = piv[0]
    lane = lax.iota(jnp.int32, LANES)

    slabs = (slab_v, slab_w)
    tgts = (tgt_v, tgt_w)
    curs = (cur_v, cur_w)
    sems = (sem_a, sem_b)

    def start_chunk(c):
        r = row0 + c * CB
        p = c % 2
        return (pltpu.async_copy(buf_hbm.at[pl.ds(r, CB)], slabs[p], sems[p]),
                pltpu.async_copy(tgt_hbm.at[pl.ds(r, CB)], tgts[p], sems[p]),
                pltpu.async_copy(cur_hbm.at[pl.ds(r, CB)], curs[p], sems[p]))

    pend = start_chunk(0)
    out_pend = None
    for c in range(NCHUNK):
        for h in pend:
            h.wait()
        if c + 1 < NCHUNK:
            pend = start_chunk(c + 1)
        slab, tgc, cuc = slabs[c % 2], tgts[c % 2], curs[c % 2]
        if out_pend is not None:
            out_pend.wait()

        def compute(i, _, slab=slab, tgc=tgc, cuc=cuc):
            el = pl.multiple_of(i * LANES, LANES) + lane
            rloc = el >> 3
            col = el & 7
            tg = plsc.load_gather(tgc, [rloc, col])
            cu = plsc.load_gather(cuc, [rloc, col])
            tx = (jnp.minimum(jnp.maximum(tg, lo), hi) - lo) * inv_dx
            ty = (jnp.minimum(jnp.maximum(cu, lo), hi) - lo) * inv_dx
            ix = jnp.minimum(tx.astype(jnp.int32), K - 2)
            iy = jnp.minimum(ty.astype(jnp.int32), K - 2)
            wx = tx - ix.astype(jnp.float32)
            wy = ty - iy.astype(jnp.float32)
            ixp = ix + 1
            iyp = iy + 1
            w00 = (1.0 - wx) * (1.0 - wy)
            w10 = wx * (1.0 - wy)
            w01 = (1.0 - wx) * wy
            w11 = wx * wy
            tau = (plsc.load_gather(tau_v, [ix, iy]) * w00
                   + plsc.load_gather(tau_v, [ixp, iy]) * w10
                   + plsc.load_gather(tau_v, [ix, iyp]) * w01
                   + plsc.load_gather(tau_v, [ixp, iyp]) * w11)
            dead = (plsc.load_gather(dead_v, [ix, iy]) * w00
                    + plsc.load_gather(dead_v, [ixp, iy]) * w10
                    + plsc.load_gather(dead_v, [ix, iyp]) * w01
                    + plsc.load_gather(dead_v, [ixp, iyp]) * w11)
            alpha = 1.0 - jnp.exp(-DT / jnp.maximum(tau, 1e-6))
            d = jnp.minimum(jnp.maximum(dead, 0.0), MAX_DELAY) / DT
            rp = wi_f - d
            i0 = rp.astype(jnp.int32)
            frac = rp - i0.astype(jnp.float32)
            c0 = jnp.bitwise_and(i0, NBUF - 1)
            c1 = jnp.bitwise_and(i0 + 1, NBUF - 1)
            g0 = plsc.load_gather(slab, [rloc, col, c0])
            g1 = plsc.load_gather(slab, [rloc, col, c1])
            s0 = jnp.where(c0 == wcol, tg, g0)
            s1 = jnp.where(c1 == wcol, tg, g1)
            s = s0 * (1.0 - frac) + s1 * frac
            plsc.store_scatter(out_v, [rloc, col], cu + alpha * (s - cu))
            return 0

        lax.fori_loop(0, CVREG, compute, 0)
        out_pend = pltpu.async_copy(
            out_v, out_hbm.at[pl.ds(row0 + c * CB, CB)], sem_o)
    out_pend.wait()


def kernel(target_pressure, buf, current_pressure, p_axis, tau_table,
           dead_table, write_idx):
    lo = p_axis[0]
    hi = p_axis[K - 1]
    inv_dx = (K - 1) / (hi - lo)
    wi_f = write_idx.astype(jnp.float32)
    pf = jnp.concatenate([jnp.stack([lo, hi, inv_dx, wi_f]),
                          jnp.zeros((12,), jnp.float32)])
    wcol = jnp.mod(write_idx, NBUF)
    pi = jnp.concatenate([wcol[None].astype(jnp.int32),
                          jnp.zeros((15,), jnp.int32)])
    return _pam_sc(target_pressure, current_pressure, buf, tau_table,
                   dead_table, pf, pi)
